# baseline (speedup 1.0000x reference)
import jax
import jax.numpy as jnp
from jax.experimental import pallas as pl
from jax.experimental.pallas import tpu as pltpu

_P = 4


def _mlp_block(x_ref, w1_ref, b1_ref, w2_ref, b2_ref,
               w3_ref, b3_ref, w4_ref, b4_ref, lo_ref, hi_ref, out_ref):
    xp = x_ref[...]
    h = jnp.dot(xp, w1_ref[...], preferred_element_type=jnp.float32)
    h = jnp.maximum(h + b1_ref[...], 0.0)
    h = jnp.dot(h, w2_ref[...], preferred_element_type=jnp.float32)
    h = jnp.maximum(h + b2_ref[...], 0.0)
    h = jnp.dot(h, w3_ref[...], preferred_element_type=jnp.float32)
    h = jnp.maximum(h + b3_ref[...], 0.0)
    v = jnp.dot(h, w4_ref[...], preferred_element_type=jnp.float32)
    v = v + b4_ref[...]
    inb = ((xp >= lo_ref[...]) & (xp <= hi_ref[...])).astype(jnp.float32)
    li = jax.lax.broadcasted_iota(jnp.int32, (4 * _P, 3 * _P), 0) // 4
    lj = jax.lax.broadcasted_iota(jnp.int32, (4 * _P, 3 * _P), 1) // 3
    sel = (li == lj).astype(jnp.float32)
    ind = jnp.dot(inb, sel, preferred_element_type=jnp.float32)
    out_ref[...] = jnp.where(ind == 4.0, v, 0.0)


def kernel(xt, bounds, W1, b1, W2, b2, W3, b3, W4, b4):
    n, d_in = xt.shape
    d_h = W1.shape[1]
    d_out = W4.shape[1]
    eye = jnp.eye(_P, dtype=jnp.float32)
    w1p = jnp.kron(eye, W1)
    w2p = jnp.kron(eye, W2)
    w3p = jnp.kron(eye, W3)
    w4p = jnp.kron(eye, W4)
    b1p = jnp.tile(b1, _P).reshape(1, -1)
    b2p = jnp.tile(b2, _P).reshape(1, -1)
    b3p = jnp.tile(b3, _P).reshape(1, -1)
    b4p = jnp.tile(b4, _P).reshape(1, -1)
    big = jnp.float32(3e38)
    lo = jnp.tile(jnp.concatenate([bounds[0], -big[None]]), _P).reshape(1, -1)
    hi = jnp.tile(jnp.concatenate([bounds[1], big[None]]), _P).reshape(1, -1)

    xr = xt.reshape(n // _P, d_in * _P)
    blk = 2048
    grid = (n // _P) // blk

    full = lambda r, c: pl.BlockSpec((r, c), lambda i: (0, 0))
    out = pl.pallas_call(
        _mlp_block,
        grid=(grid,),
        in_specs=[
            pl.BlockSpec((blk, d_in * _P), lambda i: (i, 0)),
            full(d_in * _P, d_h * _P),
            full(1, d_h * _P),
            full(d_h * _P, d_h * _P),
            full(1, d_h * _P),
            full(d_h * _P, d_h * _P),
            full(1, d_h * _P),
            full(d_h * _P, d_out * _P),
            full(1, d_out * _P),
            full(1, d_in * _P),
            full(1, d_in * _P),
        ],
        out_specs=pl.BlockSpec((blk, d_out * _P), lambda i: (i, 0)),
        out_shape=jax.ShapeDtypeStruct((n // _P, d_out * _P), jnp.float32),
        compiler_params=pltpu.CompilerParams(
            dimension_semantics=("arbitrary",)),
    )(xr, w1p, b1p, w2p, b2p, w3p, b3p, w4p, b4p, lo, hi)
    return out.reshape(n, d_out)


# speedup vs baseline: 2.2742x; 2.2742x over previous
import jax
import jax.numpy as jnp
from jax.experimental import pallas as pl
from jax.experimental.pallas import tpu as pltpu

_P = 4


def _mlp_block(x_ref, w1_ref, b1_ref, w2_ref, b2_ref, w3_ref, b3_ref,
               w4_ref, b4_ref, lo_ref, hi_ref, out_ref):
    x4 = x_ref[...]
    bp = x4.shape[0] // _P
    xp = jnp.concatenate(
        [x4[p * bp:(p + 1) * bp, :] for p in range(_P)], axis=1)
    h = jnp.dot(xp, w1_ref[...], preferred_element_type=jnp.float32)
    h = jnp.maximum(h + b1_ref[...], 0.0)
    h = jnp.dot(h, w2_ref[...], preferred_element_type=jnp.float32)
    h = jnp.maximum(h + b2_ref[...], 0.0)
    h = jnp.dot(h, w3_ref[...], preferred_element_type=jnp.float32)
    h = jnp.maximum(h + b3_ref[...], 0.0)
    v = jnp.dot(h, w4_ref[...], preferred_element_type=jnp.float32)
    v = v + b4_ref[...]
    inb = ((xp >= lo_ref[...]) & (xp <= hi_ref[...])).astype(jnp.float32)
    li = jax.lax.broadcasted_iota(jnp.int32, (4 * _P, 3 * _P), 0) // 4
    lj = jax.lax.broadcasted_iota(jnp.int32, (4 * _P, 3 * _P), 1) // 3
    sel = (li == lj).astype(jnp.float32)
    ind = jnp.dot(inb, sel, preferred_element_type=jnp.float32)
    res = jnp.where(ind == 4.0, v, 0.0)
    for p in range(_P):
        out_ref[p * bp:(p + 1) * bp, :] = res[:, 3 * p:3 * p + 3]


def kernel(xt, bounds, W1, b1, W2, b2, W3, b3, W4, b4):
    n, d_in = xt.shape
    d_h = W1.shape[1]
    d_out = W4.shape[1]
    eye = jnp.eye(_P, dtype=jnp.float32)
    w1p = jnp.kron(eye, W1)
    w2p = jnp.kron(eye, W2)
    w3p = jnp.kron(eye, W3)
    w4p = jnp.kron(eye, W4)
    b1p = jnp.tile(b1, _P).reshape(1, -1)
    b2p = jnp.tile(b2, _P).reshape(1, -1)
    b3p = jnp.tile(b3, _P).reshape(1, -1)
    b4p = jnp.tile(b4, _P).reshape(1, -1)
    big = jnp.float32(3e38)
    lo = jnp.tile(jnp.concatenate([bounds[0], -big[None]]), _P).reshape(1, -1)
    hi = jnp.tile(jnp.concatenate([bounds[1], big[None]]), _P).reshape(1, -1)

    bp = 2048
    grid = n // (bp * _P)

    full = lambda r, c: pl.BlockSpec((r, c), lambda i: (0, 0))
    out = pl.pallas_call(
        _mlp_block,
        grid=(grid,),
        in_specs=[
            pl.BlockSpec((bp * _P, d_in), lambda i: (i, 0)),
            full(d_in * _P, d_h * _P),
            full(1, d_h * _P),
            full(d_h * _P, d_h * _P),
            full(1, d_h * _P),
            full(d_h * _P, d_h * _P),
            full(1, d_h * _P),
            full(d_h * _P, d_out * _P),
            full(1, d_out * _P),
            full(1, d_in * _P),
            full(1, d_in * _P),
        ],
        out_specs=pl.BlockSpec((bp * _P, d_out), lambda i: (i, 0)),
        out_shape=jax.ShapeDtypeStruct((n, d_out), jnp.float32),
        compiler_params=pltpu.CompilerParams(
            dimension_semantics=("arbitrary",)),
    )(xt, w1p, b1p, w2p, b2p, w3p, b3p, w4p, b4p, lo, hi)
    return out
